# baseline (speedup 1.0000x reference)
import jax
import jax.numpy as jnp
from jax.experimental import pallas as pl

_MARGIN = 1.0
_N = 2048
_BLK = 256
_MSE_BLK = 128


def _body(emb_i_ref, emb_all_ref, tgt_ref, tgt_i_ref,
          o1_ref, t1_ref, o2_ref, t2_ref,
          pos_ref, neg_ref, sse1_ref, sse2_ref):
    pid = pl.program_id(0)

    e_i = emb_i_ref[...]
    e_all = emb_all_ref[...]

    dot = jax.lax.dot_general(
        e_i, e_all,
        dimension_numbers=(((1,), (1,)), ((), ())),
        preferred_element_type=jnp.float32,
        precision=jax.lax.Precision.HIGHEST,
    )

    sq_i = jnp.sum(e_i * e_i, axis=1, keepdims=True)
    sq_all = jnp.sum(e_all * e_all, axis=1, keepdims=True)
    sq_all_row = jnp.transpose(sq_all)

    d2 = jnp.maximum(sq_i + sq_all_row - 2.0 * dot, 0.0)

    tgt_all = tgt_ref[...]
    tgt_i = tgt_i_ref[...]
    same = jnp.transpose(tgt_i) == tgt_all

    row = pid * _BLK + jax.lax.broadcasted_iota(jnp.int32, (_BLK, _N), 0)
    col = jax.lax.broadcasted_iota(jnp.int32, (_BLK, _N), 1)
    offdiag = row != col

    pos_part = jnp.sum(jnp.where(same & offdiag, d2, 0.0))
    neg_vals = jnp.maximum(_MARGIN - jnp.sqrt(d2), 0.0)
    neg_part = jnp.sum(jnp.where(same, 0.0, neg_vals * neg_vals))

    r1 = o1_ref[...] - t1_ref[...]
    r2 = o2_ref[...] - t2_ref[...]
    sse1_part = jnp.sum(r1 * r1)
    sse2_part = jnp.sum(r2 * r2)

    @pl.when(pid == 0)
    def _init():
        pos_ref[...] = pos_part[None, None]
        neg_ref[...] = neg_part[None, None]
        sse1_ref[...] = sse1_part[None, None]
        sse2_ref[...] = sse2_part[None, None]

    @pl.when(pid != 0)
    def _acc():
        pos_ref[...] += pos_part[None, None]
        neg_ref[...] += neg_part[None, None]
        sse1_ref[...] += sse1_part[None, None]
        sse2_ref[...] += sse2_part[None, None]


def kernel(feature1, feature2, output1, output2, target1, target2, label):
    B, D = output1.shape
    emb = jnp.concatenate([feature1, feature2], axis=0)
    tgt = jnp.concatenate([label[0], label[1]], axis=0)[None, :]

    n_steps = _N // _BLK
    scalar = jax.ShapeDtypeStruct((1, 1), jnp.float32)

    grid_spec = pl.GridSpec(
        grid=(n_steps,),
        in_specs=[
            pl.BlockSpec((_BLK, D), lambda i: (i, 0)),
            pl.BlockSpec((_N, D), lambda i: (0, 0)),
            pl.BlockSpec((1, _N), lambda i: (0, 0)),
            pl.BlockSpec((1, _BLK), lambda i: (0, i)),
            pl.BlockSpec((_MSE_BLK, D), lambda i: (i, 0)),
            pl.BlockSpec((_MSE_BLK, D), lambda i: (i, 0)),
            pl.BlockSpec((_MSE_BLK, D), lambda i: (i, 0)),
            pl.BlockSpec((_MSE_BLK, D), lambda i: (i, 0)),
        ],
        out_specs=[
            pl.BlockSpec((1, 1), lambda i: (0, 0)),
            pl.BlockSpec((1, 1), lambda i: (0, 0)),
            pl.BlockSpec((1, 1), lambda i: (0, 0)),
            pl.BlockSpec((1, 1), lambda i: (0, 0)),
        ],
    )

    pos2, neg2, sse1, sse2 = pl.pallas_call(
        _body,
        grid_spec=grid_spec,
        out_shape=[scalar, scalar, scalar, scalar],
    )(emb, emb, tgt, tgt, output1, target1, output2, target2)

    n_pairs = jnp.float32(_N * (_N - 1) / 2)
    denom = jnp.float32(B * D)
    loss1 = sse1[0, 0] / denom
    loss2 = sse2[0, 0] / denom
    loss_mean = (0.5 * pos2[0, 0] + 0.5 * neg2[0, 0]) / n_pairs
    losses = loss_mean + (loss1 + loss2) / 2.0
    return (losses, loss1, loss2, loss_mean)


# speedup vs baseline: 1.6481x; 1.6481x over previous
import jax
import jax.numpy as jnp
from jax.experimental import pallas as pl

_MARGIN = 1.0
_N = 2048
_BLK = 256
_MSE_BLK = 128


def _body(emb_i_ref, emb_all_ref, tgt_ref, tgt_i_ref,
          o1_ref, t1_ref, o2_ref, t2_ref,
          pos_ref, neg_ref, sse1_ref, sse2_ref):
    pid = pl.program_id(0)

    e_i = emb_i_ref[...]
    e_all = emb_all_ref[...]

    dot = jax.lax.dot_general(
        e_i, e_all,
        dimension_numbers=(((1,), (1,)), ((), ())),
        preferred_element_type=jnp.float32,
        precision=jax.lax.Precision.DEFAULT,
    )

    sq_i = jnp.sum(e_i * e_i, axis=1, keepdims=True)
    sq_all = jnp.sum(e_all * e_all, axis=1, keepdims=True)
    sq_all_row = jnp.transpose(sq_all)

    d2 = jnp.maximum(sq_i + sq_all_row - 2.0 * dot, 0.0)

    tgt_all = tgt_ref[...]
    tgt_i = tgt_i_ref[...]
    same = jnp.transpose(tgt_i) == tgt_all

    row = pid * _BLK + jax.lax.broadcasted_iota(jnp.int32, (_BLK, _N), 0)
    col = jax.lax.broadcasted_iota(jnp.int32, (_BLK, _N), 1)
    offdiag = row != col

    pos_part = jnp.sum(jnp.where(same & offdiag, d2, 0.0))
    neg_vals = jnp.maximum(_MARGIN - jnp.sqrt(d2), 0.0)
    neg_part = jnp.sum(jnp.where(same, 0.0, neg_vals * neg_vals))

    r1 = o1_ref[...] - t1_ref[...]
    r2 = o2_ref[...] - t2_ref[...]
    sse1_part = jnp.sum(r1 * r1)
    sse2_part = jnp.sum(r2 * r2)

    @pl.when(pid == 0)
    def _init():
        pos_ref[...] = pos_part[None, None]
        neg_ref[...] = neg_part[None, None]
        sse1_ref[...] = sse1_part[None, None]
        sse2_ref[...] = sse2_part[None, None]

    @pl.when(pid != 0)
    def _acc():
        pos_ref[...] += pos_part[None, None]
        neg_ref[...] += neg_part[None, None]
        sse1_ref[...] += sse1_part[None, None]
        sse2_ref[...] += sse2_part[None, None]


def kernel(feature1, feature2, output1, output2, target1, target2, label):
    B, D = output1.shape
    emb = jnp.concatenate([feature1, feature2], axis=0)
    tgt = jnp.concatenate([label[0], label[1]], axis=0)[None, :]

    n_steps = _N // _BLK
    scalar = jax.ShapeDtypeStruct((1, 1), jnp.float32)

    grid_spec = pl.GridSpec(
        grid=(n_steps,),
        in_specs=[
            pl.BlockSpec((_BLK, D), lambda i: (i, 0)),
            pl.BlockSpec((_N, D), lambda i: (0, 0)),
            pl.BlockSpec((1, _N), lambda i: (0, 0)),
            pl.BlockSpec((1, _BLK), lambda i: (0, i)),
            pl.BlockSpec((_MSE_BLK, D), lambda i: (i, 0)),
            pl.BlockSpec((_MSE_BLK, D), lambda i: (i, 0)),
            pl.BlockSpec((_MSE_BLK, D), lambda i: (i, 0)),
            pl.BlockSpec((_MSE_BLK, D), lambda i: (i, 0)),
        ],
        out_specs=[
            pl.BlockSpec((1, 1), lambda i: (0, 0)),
            pl.BlockSpec((1, 1), lambda i: (0, 0)),
            pl.BlockSpec((1, 1), lambda i: (0, 0)),
            pl.BlockSpec((1, 1), lambda i: (0, 0)),
        ],
    )

    pos2, neg2, sse1, sse2 = pl.pallas_call(
        _body,
        grid_spec=grid_spec,
        out_shape=[scalar, scalar, scalar, scalar],
    )(emb, emb, tgt, tgt, output1, target1, output2, target2)

    n_pairs = jnp.float32(_N * (_N - 1) / 2)
    denom = jnp.float32(B * D)
    loss1 = sse1[0, 0] / denom
    loss2 = sse2[0, 0] / denom
    loss_mean = (0.5 * pos2[0, 0] + 0.5 * neg2[0, 0]) / n_pairs
    losses = loss_mean + (loss1 + loss2) / 2.0
    return (losses, loss1, loss2, loss_mean)


# speedup vs baseline: 1.8214x; 1.1051x over previous
import jax
import jax.numpy as jnp
from jax.experimental import pallas as pl

_MARGIN = 1.0
_N = 2048
_BLK = 256
_MSE_BLK = 128


def _body(emb_i_ref, emb_all_ref, tgt_ref, tgt_i_ref,
          o1_ref, t1_ref, o2_ref, t2_ref,
          pair_ref, sse1_ref, sse2_ref):
    pid = pl.program_id(0)

    e_i = emb_i_ref[...]
    e_all = emb_all_ref[...]

    dot = jax.lax.dot_general(
        e_i, e_all,
        dimension_numbers=(((1,), (1,)), ((), ())),
        preferred_element_type=jnp.float32,
        precision=jax.lax.Precision.DEFAULT,
    )

    sq_i = jnp.sum(e_i * e_i, axis=1, keepdims=True)
    sq_all = jnp.sum(e_all * e_all, axis=1, keepdims=True)
    sq_all_row = jnp.transpose(sq_all)

    d2 = jnp.maximum(sq_i + sq_all_row - 2.0 * dot, 0.0)

    tgt_all = tgt_ref[...]
    tgt_i = tgt_i_ref[...]
    same = jnp.transpose(tgt_i) == tgt_all

    neg_vals = jnp.maximum(_MARGIN - jnp.sqrt(d2), 0.0)
    pair_part = jnp.sum(jnp.where(same, d2, neg_vals * neg_vals))

    r1 = o1_ref[...] - t1_ref[...]
    r2 = o2_ref[...] - t2_ref[...]
    sse1_part = jnp.sum(r1 * r1)
    sse2_part = jnp.sum(r2 * r2)

    @pl.when(pid == 0)
    def _init():
        pair_ref[...] = pair_part[None, None]
        sse1_ref[...] = sse1_part[None, None]
        sse2_ref[...] = sse2_part[None, None]

    @pl.when(pid != 0)
    def _acc():
        pair_ref[...] += pair_part[None, None]
        sse1_ref[...] += sse1_part[None, None]
        sse2_ref[...] += sse2_part[None, None]


def kernel(feature1, feature2, output1, output2, target1, target2, label):
    B, D = output1.shape
    emb = jnp.concatenate([feature1, feature2], axis=0)
    tgt = jnp.concatenate([label[0], label[1]], axis=0)[None, :]

    n_steps = _N // _BLK
    scalar = jax.ShapeDtypeStruct((1, 1), jnp.float32)

    grid_spec = pl.GridSpec(
        grid=(n_steps,),
        in_specs=[
            pl.BlockSpec((_BLK, D), lambda i: (i, 0)),
            pl.BlockSpec((_N, D), lambda i: (0, 0)),
            pl.BlockSpec((1, _N), lambda i: (0, 0)),
            pl.BlockSpec((1, _BLK), lambda i: (0, i)),
            pl.BlockSpec((_MSE_BLK, D), lambda i: (i, 0)),
            pl.BlockSpec((_MSE_BLK, D), lambda i: (i, 0)),
            pl.BlockSpec((_MSE_BLK, D), lambda i: (i, 0)),
            pl.BlockSpec((_MSE_BLK, D), lambda i: (i, 0)),
        ],
        out_specs=[
            pl.BlockSpec((1, 1), lambda i: (0, 0)),
            pl.BlockSpec((1, 1), lambda i: (0, 0)),
            pl.BlockSpec((1, 1), lambda i: (0, 0)),
        ],
    )

    pair2, sse1, sse2 = pl.pallas_call(
        _body,
        grid_spec=grid_spec,
        out_shape=[scalar, scalar, scalar],
    )(emb, emb, tgt, tgt, output1, target1, output2, target2)

    n_pairs = jnp.float32(_N * (_N - 1) / 2)
    denom = jnp.float32(B * D)
    loss1 = sse1[0, 0] / denom
    loss2 = sse2[0, 0] / denom
    loss_mean = 0.5 * pair2[0, 0] / n_pairs
    losses = loss_mean + (loss1 + loss2) / 2.0
    return (losses, loss1, loss2, loss_mean)
